# baseline (speedup 1.0000x reference)
import functools

import jax
import jax.numpy as jnp
from jax import lax
from jax.experimental import pallas as pl
from jax.experimental.pallas import tpu as pltpu
from jax.experimental.pallas import tpu_sc as plsc

_N = 10000
_E = 320000
_D = 128
_H = 64
_C = 2

_NC = 2
_NS = 16
_NW = _NC * _NS
_EPT = _E // _NW
_CH = 80
_NCHUNK = _EPT // _CH
_NPAD = 10240
_ROWS = _NPAD // _NS

_NB = 5
_NBLK = _NCHUNK // _NB
_NPAIR = (_NBLK - 1) // 2


def _make_edge_scatter(W, gather):
  mesh = plsc.VectorSubcoreMesh(core_axis_name="c", subcore_axis_name="s")
  lanes = W // 16

  @functools.partial(
      pl.kernel,
      out_type=jax.ShapeDtypeStruct((_NC, _NPAD, W), jnp.float32),
      mesh=mesh,
      compiler_params=pltpu.CompilerParams(use_tc_tiling_on_sc=False),
      scratch_types=[
          pltpu.VMEM((_NCHUNK, _CH), jnp.int32),
          pltpu.VMEM((_NCHUNK, _CH), jnp.int32),
          pltpu.VMEM((2, _NB, _CH, W), jnp.float32),
          pltpu.VMEM_SHARED((_NPAD, W), jnp.float32),
          pltpu.SemaphoreType.DMA,
          pltpu.SemaphoreType.DMA,
          pltpu.SemaphoreType.DMA,
      ],
  )
  def k(ei3_hbm, rows_hbm, out_hbm,
        sidx2, didx2, rows, acc, gsem, ssem0, ssem1):
    c = lax.axis_index("c")
    s = lax.axis_index("s")
    tile = c * _NS + s
    r0 = s * _ROWS

    def zfill(j, carry):
      rows[1, 0, j // lanes, pl.ds((j % lanes) * 16, 16)] = (
          jnp.zeros((16,), jnp.float32))
      return carry

    lax.fori_loop(0, _CH * lanes, zfill, 0)
    zh = [pltpu.async_copy(rows.at[1, 0], acc.at[pl.ds(r0 + i * _CH, _CH)],
                           gsem)
          for i in range(_ROWS // _CH)]
    cb = tile * _NCHUNK
    if gather:
      pltpu.sync_copy(ei3_hbm.at[0, pl.ds(cb, _NCHUNK)], sidx2)
    else:
      pltpu.sync_copy(rows_hbm, rows.at[0, 0])
    pltpu.sync_copy(ei3_hbm.at[1, pl.ds(cb, _NCHUNK)], didx2)
    for h in zh:
      h.wait()
    plsc.subcore_barrier()

    def gather_block(setp, q0):
      hs = [pltpu.async_copy(rows_hbm.at[sidx2.at[q0 + b]],
                             rows.at[setp, b], gsem)
            for b in range(_NB)]
      for h in hs:
        h.wait()

    def fire_scatters(setp, q0, sem):
      for b in range(_NB):
        pltpu.async_copy(rows.at[setp, b], acc.at[didx2.at[q0 + b]],
                         sem, add=True)

    def drain_scatters(setp, sem):
      dummy = rows_hbm.at[pl.ds(0, _CH)] if gather else rows_hbm
      for b in range(_NB):
        pltpu.make_async_copy(dummy, rows.at[setp, b], sem).wait()

    if gather:
      def pair(i, carry):
        qa = (2 * i) * _NB
        qb = (2 * i + 1) * _NB

        @pl.when(i > 0)
        def _():
          drain_scatters(0, ssem0)
        gather_block(0, qa)
        fire_scatters(0, qa, ssem0)

        @pl.when(i > 0)
        def _():
          drain_scatters(1, ssem1)
        gather_block(1, qb)
        fire_scatters(1, qb, ssem1)
        return carry

      lax.fori_loop(0, _NPAIR, pair, 0)
      qe = (_NBLK - 1) * _NB
      drain_scatters(0, ssem0)
      gather_block(0, qe)
      fire_scatters(0, qe, ssem0)
      drain_scatters(0, ssem0)
      drain_scatters(1, ssem1)
    else:
      def blockd(i, carry):
        for b in range(_NB):
          pltpu.async_copy(rows.at[0, 0], acc.at[didx2.at[i * _NB + b]],
                           ssem0, add=True)

        @pl.when(i > 0)
        def _():
          drain_scatters(0, ssem0)
        return carry

      lax.fori_loop(0, _NBLK, blockd, 0)
      drain_scatters(0, ssem0)

    plsc.subcore_barrier()
    pltpu.sync_copy(acc.at[pl.ds(r0, _ROWS)], out_hbm.at[c, pl.ds(r0, _ROWS)])

  return k


_deg_scatter = _make_edge_scatter(16, gather=False)
_pass1_scatter = _make_edge_scatter(_H, gather=True)
_pass2_scatter = _make_edge_scatter(16, gather=True)

_BN = 2000
_G = _N // _BN


def _tc_mm_body(x_ref, w1, b1r, w2, b2r, wg1, xw_ref, mlp_ref):
  xv = x_ref[...]
  xw_ref[...] = jnp.dot(xv, wg1[...], preferred_element_type=jnp.float32)
  mh = jnp.maximum(
      jnp.dot(xv, w1[...], preferred_element_type=jnp.float32) + b1r[...], 0.0)
  mlp_ref[...] = jnp.dot(mh, w2[...], preferred_element_type=jnp.float32) + b2r[...]


_tc_mm = pl.pallas_call(
    _tc_mm_body,
    grid=(_G,),
    in_specs=[
        pl.BlockSpec((_BN, _D), lambda i: (i, 0)),
        pl.BlockSpec((_D, _H), lambda i: (0, 0)),
        pl.BlockSpec((1, _H), lambda i: (0, 0)),
        pl.BlockSpec((_H, _C), lambda i: (0, 0)),
        pl.BlockSpec((1, _C), lambda i: (0, 0)),
        pl.BlockSpec((_D, _H), lambda i: (0, 0)),
    ],
    out_specs=[
        pl.BlockSpec((_BN, _H), lambda i: (i, 0)),
        pl.BlockSpec((_BN, _C), lambda i: (i, 0)),
    ],
    out_shape=[
        jax.ShapeDtypeStruct((_N, _H), jnp.float32),
        jax.ShapeDtypeStruct((_N, _C), jnp.float32),
    ],
)


def _tc_scale_body(deg1_ref, xw_ref, y_ref, dinv_ref):
  deg1 = deg1_ref[...] + 1.0
  dinv = lax.rsqrt(jnp.maximum(deg1, 1e-12))
  y_ref[...] = xw_ref[...] * dinv
  dinv_ref[...] = dinv


_tc_scale = pl.pallas_call(
    _tc_scale_body,
    grid=(_G,),
    in_specs=[
        pl.BlockSpec((_BN, 1), lambda i: (i, 0)),
        pl.BlockSpec((_BN, _H), lambda i: (i, 0)),
    ],
    out_specs=[
        pl.BlockSpec((_BN, _H), lambda i: (i, 0)),
        pl.BlockSpec((_BN, 1), lambda i: (i, 0)),
    ],
    out_shape=[
        jax.ShapeDtypeStruct((_N, _H), jnp.float32),
        jax.ShapeDtypeStruct((_N, 1), jnp.float32),
    ],
)


def _tc_mid_body(hps_ref, y_ref, dinv_ref, bg1r, wg2, up_ref, us_ref):
  hpre = hps_ref[...] + y_ref[...]
  h = hpre * dinv_ref[...] + bg1r[...]
  hr = jnp.maximum(h, 0.0)
  u = jnp.dot(hr, wg2[...], preferred_element_type=jnp.float32)
  us = u * dinv_ref[...]
  us_ref[...] = us
  up_ref[...] = jnp.concatenate(
      [us, jnp.zeros((_BN, 16 - _C), jnp.float32)], axis=1)


_tc_mid = pl.pallas_call(
    _tc_mid_body,
    grid=(_G,),
    in_specs=[
        pl.BlockSpec((_BN, _H), lambda i: (i, 0)),
        pl.BlockSpec((_BN, _H), lambda i: (i, 0)),
        pl.BlockSpec((_BN, 1), lambda i: (i, 0)),
        pl.BlockSpec((1, _H), lambda i: (0, 0)),
        pl.BlockSpec((_H, _C), lambda i: (0, 0)),
    ],
    out_specs=[
        pl.BlockSpec((_BN, 16), lambda i: (i, 0)),
        pl.BlockSpec((_BN, _C), lambda i: (i, 0)),
    ],
    out_shape=[
        jax.ShapeDtypeStruct((_N, 16), jnp.float32),
        jax.ShapeDtypeStruct((_N, _C), jnp.float32),
    ],
)


def _tc_final_body(spc_ref, us_ref, dinv_ref, mlp_ref, bg2r, out_ref):
  o = mlp_ref[...] + (spc_ref[...] + us_ref[...]) * dinv_ref[...] + bg2r[...]
  m = jnp.max(o, axis=1, keepdims=True)
  lse = m + jnp.log(jnp.sum(jnp.exp(o - m), axis=1, keepdims=True))
  out_ref[...] = o - lse


_tc_final = pl.pallas_call(
    _tc_final_body,
    grid=(_G,),
    in_specs=[
        pl.BlockSpec((_BN, _C), lambda i: (i, 0)),
        pl.BlockSpec((_BN, _C), lambda i: (i, 0)),
        pl.BlockSpec((_BN, 1), lambda i: (i, 0)),
        pl.BlockSpec((_BN, _C), lambda i: (i, 0)),
        pl.BlockSpec((1, _C), lambda i: (0, 0)),
    ],
    out_specs=pl.BlockSpec((_BN, _C), lambda i: (i, 0)),
    out_shape=jax.ShapeDtypeStruct((_N, _C), jnp.float32),
)


def kernel(x, edge_index, W1, b1, W2, b2, Wg1, bg1, Wg2, bg2):
  ei3 = edge_index.reshape(2, _E // _CH, _CH)
  ones = jnp.ones((_CH, 16), jnp.float32)
  b1r = b1.reshape(1, _H)
  b2r = b2.reshape(1, _C)
  bg1r = bg1.reshape(1, _H)
  bg2r = bg2.reshape(1, _C)

  degp = _deg_scatter(ei3, ones)
  deg1 = degp[0, :, 0:1] + degp[1, :, 0:1]
  xw, mlp = _tc_mm(x, W1, b1r, W2, b2r, Wg1)
  y, dinv = _tc_scale(deg1[:_N], xw)
  hp = _pass1_scatter(ei3, y)
  hps = hp[0, :_N] + hp[1, :_N]
  up, us = _tc_mid(hps, y, dinv, bg1r, Wg2)
  sp = _pass2_scatter(ei3, up)
  spc = sp[0, :_N, 0:_C] + sp[1, :_N, 0:_C]
  return _tc_final(spc, us, dinv, mlp, bg2r)


# speedup vs baseline: 1.0783x; 1.0783x over previous
import functools

import jax
import jax.numpy as jnp
from jax import lax
from jax.experimental import pallas as pl
from jax.experimental.pallas import tpu as pltpu
from jax.experimental.pallas import tpu_sc as plsc

_N = 10000
_E = 320000
_D = 128
_H = 64
_C = 2

_NC = 2
_NS = 16
_NW = _NC * _NS
_EPT = _E // _NW
_CH = 80
_NCHUNK = _EPT // _CH
_NPAD = 10240
_ROWS = _NPAD // _NS

_NB = 5
_NBLK = _NCHUNK // _NB
_NPAIR = (_NBLK - 1) // 2


def _make_edge_scatter(W, gather):
  mesh = plsc.VectorSubcoreMesh(core_axis_name="c", subcore_axis_name="s")
  lanes = W // 16

  @functools.partial(
      pl.kernel,
      out_type=jax.ShapeDtypeStruct((_NC, _NPAD, W), jnp.float32),
      mesh=mesh,
      compiler_params=pltpu.CompilerParams(use_tc_tiling_on_sc=False),
      scratch_types=[
          pltpu.VMEM((_NCHUNK, _CH), jnp.int32),
          pltpu.VMEM((_NCHUNK, _CH), jnp.int32),
          pltpu.VMEM((2, _NB, _CH, W), jnp.float32),
          pltpu.VMEM_SHARED((_NPAD, W), jnp.float32),
          pltpu.SemaphoreType.DMA,
          pltpu.SemaphoreType.DMA,
          pltpu.SemaphoreType.DMA,
      ],
  )
  def k(ei3_hbm, rows_hbm, out_hbm,
        sidx2, didx2, rows, acc, gsem, ssem0, ssem1):
    c = lax.axis_index("c")
    s = lax.axis_index("s")
    tile = c * _NS + s
    r0 = s * _ROWS

    def zfill(j, carry):
      rows[1, 0, j // lanes, pl.ds((j % lanes) * 16, 16)] = (
          jnp.zeros((16,), jnp.float32))
      return carry

    lax.fori_loop(0, _CH * lanes, zfill, 0)
    zh = [pltpu.async_copy(rows.at[1, 0], acc.at[pl.ds(r0 + i * _CH, _CH)],
                           gsem)
          for i in range(_ROWS // _CH)]
    cb = tile * _NCHUNK
    if gather:
      pltpu.sync_copy(ei3_hbm.at[0, pl.ds(cb, _NCHUNK)], sidx2)
    else:
      pltpu.sync_copy(rows_hbm, rows.at[0, 0])
    pltpu.sync_copy(ei3_hbm.at[1, pl.ds(cb, _NCHUNK)], didx2)
    for h in zh:
      h.wait()
    plsc.subcore_barrier()

    def gather_block(setp, q0):
      hs = [pltpu.async_copy(rows_hbm.at[sidx2.at[q0 + b]],
                             rows.at[setp, b], gsem)
            for b in range(_NB)]
      for h in hs:
        h.wait()

    def fire_scatters(setp, q0, sem):
      for b in range(_NB):
        pltpu.async_copy(rows.at[setp, b], acc.at[didx2.at[q0 + b]],
                         sem, add=True)

    def drain_scatters(setp, sem):
      dummy = rows_hbm.at[pl.ds(0, _CH)] if gather else rows_hbm
      for b in range(_NB):
        pltpu.make_async_copy(dummy, rows.at[setp, b], sem).wait()

    if gather:
      def pair(i, carry):
        qa = (2 * i) * _NB
        qb = (2 * i + 1) * _NB

        @pl.when(i > 0)
        def _():
          drain_scatters(0, ssem0)
        gather_block(0, qa)
        fire_scatters(0, qa, ssem0)

        @pl.when(i > 0)
        def _():
          drain_scatters(1, ssem1)
        gather_block(1, qb)
        fire_scatters(1, qb, ssem1)
        return carry

      lax.fori_loop(0, _NPAIR, pair, 0)
      qe = (_NBLK - 1) * _NB
      drain_scatters(0, ssem0)
      gather_block(0, qe)
      fire_scatters(0, qe, ssem0)
      drain_scatters(0, ssem0)
      drain_scatters(1, ssem1)
    else:
      def blockd(i, carry):
        for b in range(_NB):
          pltpu.async_copy(rows.at[0, 0], acc.at[didx2.at[i * _NB + b]],
                           ssem0, add=True)

        @pl.when(i > 0)
        def _():
          drain_scatters(0, ssem0)
        return carry

      lax.fori_loop(0, _NBLK, blockd, 0)
      drain_scatters(0, ssem0)

    plsc.subcore_barrier()
    pltpu.sync_copy(acc.at[pl.ds(r0, _ROWS)], out_hbm.at[c, pl.ds(r0, _ROWS)])

  return k


_deg_scatter = _make_edge_scatter(16, gather=False)
_pass1_scatter = _make_edge_scatter(_H, gather=True)
_pass2_scatter = _make_edge_scatter(16, gather=True)

_BN = 2000
_G = _N // _BN


def _tc_mm_body(x_ref, w1, b1r, w2, b2r, wg1, xw_ref, mlp_ref):
  xv = x_ref[...]
  xw_ref[...] = jnp.dot(xv, wg1[...], preferred_element_type=jnp.float32)
  mh = jnp.maximum(
      jnp.dot(xv, w1[...], preferred_element_type=jnp.float32) + b1r[...], 0.0)
  mlp_ref[...] = jnp.dot(mh, w2[...], preferred_element_type=jnp.float32) + b2r[...]


_tc_mm = pl.pallas_call(
    _tc_mm_body,
    grid=(_G,),
    in_specs=[
        pl.BlockSpec((_BN, _D), lambda i: (i, 0)),
        pl.BlockSpec((_D, _H), lambda i: (0, 0)),
        pl.BlockSpec((1, _H), lambda i: (0, 0)),
        pl.BlockSpec((_H, _C), lambda i: (0, 0)),
        pl.BlockSpec((1, _C), lambda i: (0, 0)),
        pl.BlockSpec((_D, _H), lambda i: (0, 0)),
    ],
    out_specs=[
        pl.BlockSpec((_BN, _H), lambda i: (i, 0)),
        pl.BlockSpec((_BN, _C), lambda i: (i, 0)),
    ],
    out_shape=[
        jax.ShapeDtypeStruct((_N, _H), jnp.float32),
        jax.ShapeDtypeStruct((_N, _C), jnp.float32),
    ],
)


def _tc_scale_body(degp_ref, xw_ref, y_ref, dinv_ref):
  deg1 = degp_ref[0, :, 0:1] + degp_ref[1, :, 0:1] + 1.0
  dinv = lax.rsqrt(jnp.maximum(deg1, 1e-12))
  y_ref[...] = xw_ref[...] * dinv
  dinv_ref[...] = dinv


_tc_scale = pl.pallas_call(
    _tc_scale_body,
    grid=(_G,),
    in_specs=[
        pl.BlockSpec((2, _BN, 16), lambda i: (0, i, 0)),
        pl.BlockSpec((_BN, _H), lambda i: (i, 0)),
    ],
    out_specs=[
        pl.BlockSpec((_BN, _H), lambda i: (i, 0)),
        pl.BlockSpec((_BN, 1), lambda i: (i, 0)),
    ],
    out_shape=[
        jax.ShapeDtypeStruct((_N, _H), jnp.float32),
        jax.ShapeDtypeStruct((_N, 1), jnp.float32),
    ],
)


def _tc_mid_body(hp_ref, y_ref, dinv_ref, bg1r, wg2, up_ref, us_ref):
  hpre = hp_ref[0] + hp_ref[1] + y_ref[...]
  h = hpre * dinv_ref[...] + bg1r[...]
  hr = jnp.maximum(h, 0.0)
  u = jnp.dot(hr, wg2[...], preferred_element_type=jnp.float32)
  us = u * dinv_ref[...]
  us_ref[...] = us
  up_ref[...] = jnp.concatenate(
      [us, jnp.zeros((_BN, 16 - _C), jnp.float32)], axis=1)


_tc_mid = pl.pallas_call(
    _tc_mid_body,
    grid=(_G,),
    in_specs=[
        pl.BlockSpec((2, _BN, _H), lambda i: (0, i, 0)),
        pl.BlockSpec((_BN, _H), lambda i: (i, 0)),
        pl.BlockSpec((_BN, 1), lambda i: (i, 0)),
        pl.BlockSpec((1, _H), lambda i: (0, 0)),
        pl.BlockSpec((_H, _C), lambda i: (0, 0)),
    ],
    out_specs=[
        pl.BlockSpec((_BN, 16), lambda i: (i, 0)),
        pl.BlockSpec((_BN, _C), lambda i: (i, 0)),
    ],
    out_shape=[
        jax.ShapeDtypeStruct((_N, 16), jnp.float32),
        jax.ShapeDtypeStruct((_N, _C), jnp.float32),
    ],
)


def _tc_final_body(sp_ref, us_ref, dinv_ref, mlp_ref, bg2r, out_ref):
  spc = sp_ref[0, :, 0:_C] + sp_ref[1, :, 0:_C]
  o = mlp_ref[...] + (spc + us_ref[...]) * dinv_ref[...] + bg2r[...]
  m = jnp.max(o, axis=1, keepdims=True)
  lse = m + jnp.log(jnp.sum(jnp.exp(o - m), axis=1, keepdims=True))
  out_ref[...] = o - lse


_tc_final = pl.pallas_call(
    _tc_final_body,
    grid=(_G,),
    in_specs=[
        pl.BlockSpec((2, _BN, 16), lambda i: (0, i, 0)),
        pl.BlockSpec((_BN, _C), lambda i: (i, 0)),
        pl.BlockSpec((_BN, 1), lambda i: (i, 0)),
        pl.BlockSpec((_BN, _C), lambda i: (i, 0)),
        pl.BlockSpec((1, _C), lambda i: (0, 0)),
    ],
    out_specs=pl.BlockSpec((_BN, _C), lambda i: (i, 0)),
    out_shape=jax.ShapeDtypeStruct((_N, _C), jnp.float32),
)


def kernel(x, edge_index, W1, b1, W2, b2, Wg1, bg1, Wg2, bg2):
  ei3 = edge_index.reshape(2, _E // _CH, _CH)
  ones = jnp.ones((_CH, 16), jnp.float32)
  b1r = b1.reshape(1, _H)
  b2r = b2.reshape(1, _C)
  bg1r = bg1.reshape(1, _H)
  bg2r = bg2.reshape(1, _C)

  degp = _deg_scatter(ei3, ones)
  xw, mlp = _tc_mm(x, W1, b1r, W2, b2r, Wg1)
  y, dinv = _tc_scale(degp, xw)
  hp = _pass1_scatter(ei3, y)
  up, us = _tc_mid(hp, y, dinv, bg1r, Wg2)
  sp = _pass2_scatter(ei3, up)
  return _tc_final(sp, us, dinv, mlp, bg2r)
